# Initial kernel scaffold; baseline (speedup 1.0000x reference)
import jax
import jax.numpy as jnp
from jax.experimental import pallas as pl


def kernel(x, params):
    raise NotImplementedError("write your pallas kernel here")


# speedup vs baseline: 22.4334x; 22.4334x over previous
import functools
import math

import jax
import jax.numpy as jnp
from jax import lax
from jax.experimental import pallas as pl
from jax.experimental.pallas import tpu as pltpu

_K = 20
_BN_EPS = 1e-5
_GRAD_WIDTH = 10.0
_SQRT_2PI = math.sqrt(2.0 * math.pi)


def _spike(v):
    xc = jnp.clip(v, -10.0, 10.0)
    gaussian = jnp.exp(-xc * xc / 2.0) / _SQRT_2PI
    sigmoid = jax.nn.sigmoid(_GRAD_WIDTH * xc)
    return 0.5 * gaussian + 0.5 * sigmoid


def _edge_kernel(ht_ref, h_ref, wsum_ref, wa_ref, bias_ref, tb_ref, out_ref, *, n, k):
    ht = ht_ref[...]
    h = h_ref[...]
    inner = -2.0 * jnp.dot(ht, h, preferred_element_type=jnp.float32)
    xx_col = jnp.sum(ht * ht, axis=1, keepdims=True)
    xx_row = jnp.sum(h * h, axis=0, keepdims=True)
    vals = -xx_col - inner - xx_row
    iota = lax.broadcasted_iota(jnp.int32, (n, n), 1)

    ut = jnp.dot(ht, wsum_ref[...], preferred_element_type=jnp.float32)
    vbt = jnp.dot(ht, wa_ref[...], preferred_element_type=jnp.float32) - bias_ref[...]
    tb = tb_ref[...]

    hmax = None
    for _ in range(k):
        mx = jnp.max(vals, axis=1, keepdims=True)
        aj = jnp.min(jnp.where(vals == mx, iota, n), axis=1, keepdims=True)
        sel = iota == aj
        vals = jnp.where(sel, -jnp.inf, vals)
        onehot = jnp.where(sel, 1.0, 0.0)
        g = jnp.dot(onehot, ut, preferred_element_type=jnp.float32)
        y = g - vbt
        y = jnp.where(y >= 0, y, 0.2 * y)
        sp = _spike(y - tb)
        hmax = sp if hmax is None else jnp.maximum(hmax, sp)
    out_ref[...] = hmax


def _edge_layer(ht, h, wsum_t, wa_t, bias_row, tb_row):
    bsz, n, c = ht.shape
    cout = wsum_t.shape[1]
    return pl.pallas_call(
        functools.partial(_edge_kernel, n=n, k=_K),
        grid=(bsz,),
        in_specs=[
            pl.BlockSpec((None, n, c), lambda b: (b, 0, 0)),
            pl.BlockSpec((None, c, n), lambda b: (b, 0, 0)),
            pl.BlockSpec((c, cout), lambda b: (0, 0)),
            pl.BlockSpec((c, cout), lambda b: (0, 0)),
            pl.BlockSpec((1, cout), lambda b: (0, 0)),
            pl.BlockSpec((1, cout), lambda b: (0, 0)),
        ],
        out_specs=pl.BlockSpec((None, n, cout), lambda b: (b, 0, 0)),
        out_shape=jax.ShapeDtypeStruct((bsz, n, cout), jnp.float32),
    )(ht, h, wsum_t, wa_t, bias_row, tb_row)


def _final_kernel(mt_ref, wms_ref, bias_ref, tb_ref, ta_ref, md_ref, rd_ref,
                  w_ref, out_ref):
    emb = jnp.dot(mt_ref[...], wms_ref[...], preferred_element_type=jnp.float32)
    emb = emb + bias_ref[...]
    emb = jnp.where(emb >= 0, emb, 0.2 * emb)
    g = jnp.max(emb, axis=0, keepdims=True)
    tb = tb_ref[...]
    ta = ta_ref[...]
    md = md_ref[...]
    rd = rd_ref[...]
    m = g
    sp0 = _spike(m - tb)
    m = m * (1.0 - sp0)
    rf = sp0
    th = tb + ((tb + ta * sp0) - tb) * 0.95
    m = m * md * (1.0 - rf)
    sp1 = _spike(m - th)
    m = m * (1.0 - sp1)
    rf = rf * rd + sp1
    th = tb + ((th + ta * sp1) - tb) * 0.95
    m = m * md * (1.0 - rf)
    sp2 = _spike(m - th)
    out_ref[...] = w_ref[0] * sp0 + w_ref[1] * sp1 + w_ref[2] * sp2


def _final_layer(mt, wms_t, bias_row, tb, ta, md, rd, w3):
    bsz, n, cm = mt.shape
    e = wms_t.shape[1]
    row = lambda b: (0, 0)
    out = pl.pallas_call(
        _final_kernel,
        grid=(bsz,),
        in_specs=[
            pl.BlockSpec((None, n, cm), lambda b: (b, 0, 0)),
            pl.BlockSpec((cm, e), row),
            pl.BlockSpec((1, e), row),
            pl.BlockSpec((1, e), row),
            pl.BlockSpec((1, e), row),
            pl.BlockSpec((1, e), row),
            pl.BlockSpec((1, e), row),
            pl.BlockSpec(memory_space=pltpu.SMEM),
        ],
        out_specs=pl.BlockSpec((None, 1, e), lambda b: (b, 0, 0)),
        out_shape=jax.ShapeDtypeStruct((bsz, 1, e), jnp.float32),
    )(mt, wms_t, bias_row, tb, ta, md, rd, w3)
    return out[:, 0, :]


def kernel(x, params):
    scale = 1.0 / math.sqrt(1.0 + _BN_EPS)
    ht = jnp.transpose(x, (0, 2, 1))
    h = x
    feats = []
    for i in range(4):
        w = params['w%d' % i]
        c = w.shape[1] // 2
        sg = (params['g%d' % i] * scale)[:, None]
        wa_t = jnp.transpose(w[:, :c] * sg)
        wsum_t = jnp.transpose((w[:, :c] + w[:, c:]) * sg)
        ht = _edge_layer(ht, h, wsum_t, wa_t,
                         params['b%d' % i][None, :],
                         params['tb%d' % i][None, :])
        h = jnp.transpose(ht, (0, 2, 1))
        feats.append(ht)
    mt = jnp.concatenate(feats, axis=-1)
    wms_t = jnp.transpose(params['wms'] * (params['gms'] * scale)[:, None])
    w3 = jax.nn.softmax(params['tw'])
    return _final_layer(
        mt, wms_t,
        params['bms'][None, :],
        params['tb4'][None, :],
        jnp.clip(params['ta4'], 0.001, 0.1)[None, :],
        jnp.clip(params['md4'], 0.1, 0.99)[None, :],
        jnp.clip(params['rd4'], 0.1, 0.95)[None, :],
        w3,
    )


# speedup vs baseline: 30.6834x; 1.3678x over previous
import functools
import math

import jax
import jax.numpy as jnp
from jax import lax
from jax.experimental import pallas as pl
from jax.experimental.pallas import tpu as pltpu

_K = 20
_BN_EPS = 1e-5
_GRAD_WIDTH = 10.0
_SQRT_2PI = math.sqrt(2.0 * math.pi)
_SPIKE_PEAK = 0.4154990014554293


def _spike(v):
    xc = jnp.clip(v, -10.0, 10.0)
    gaussian = jnp.exp(-xc * xc / 2.0) / _SQRT_2PI
    sigmoid = jax.nn.sigmoid(_GRAD_WIDTH * xc)
    return 0.5 * gaussian + 0.5 * sigmoid


def _edge_kernel(ht_ref, h_ref, wsum_ref, wa_ref, bias_ref, tb_ref, out_ref, *, n, k):
    ht = ht_ref[...]
    h = h_ref[...]
    inner = -2.0 * jnp.dot(ht, h, preferred_element_type=jnp.float32)
    xx_col = jnp.sum(ht * ht, axis=1, keepdims=True)
    xx_row = jnp.sum(h * h, axis=0, keepdims=True)
    vals = -xx_col - inner - xx_row
    iota = lax.broadcasted_iota(jnp.int32, (n, n), 1)

    ut = jnp.dot(ht, wsum_ref[...], preferred_element_type=jnp.float32)
    vbt = jnp.dot(ht, wa_ref[...], preferred_element_type=jnp.float32) - bias_ref[...]
    tb = tb_ref[...]

    lo = None
    hi = None
    for _ in range(k):
        mx = jnp.max(vals, axis=1, keepdims=True)
        aj = jnp.min(jnp.where(vals == mx, iota, n), axis=1, keepdims=True)
        sel = iota == aj
        vals = jnp.where(sel, -jnp.inf, vals)
        onehot = jnp.where(sel, 1.0, 0.0)
        g = jnp.dot(onehot, ut, preferred_element_type=jnp.float32)
        y = g - vbt
        y = jnp.where(y >= 0, y, 0.2 * y)
        v = y - tb
        lo_j = jnp.where(v <= _SPIKE_PEAK, v, -jnp.inf)
        hi_j = jnp.where(v <= _SPIKE_PEAK, jnp.inf, v)
        lo = lo_j if lo is None else jnp.maximum(lo, lo_j)
        hi = hi_j if hi is None else jnp.minimum(hi, hi_j)
    sp_lo = _spike(lo)
    sp_hi = jnp.where(hi == jnp.inf, 0.0, _spike(hi))
    out_ref[...] = jnp.maximum(sp_lo, sp_hi)


def _edge_layer(ht, h, wsum_t, wa_t, bias_row, tb_row):
    bsz, n, c = ht.shape
    cout = wsum_t.shape[1]
    return pl.pallas_call(
        functools.partial(_edge_kernel, n=n, k=_K),
        grid=(bsz,),
        in_specs=[
            pl.BlockSpec((None, n, c), lambda b: (b, 0, 0)),
            pl.BlockSpec((None, c, n), lambda b: (b, 0, 0)),
            pl.BlockSpec((c, cout), lambda b: (0, 0)),
            pl.BlockSpec((c, cout), lambda b: (0, 0)),
            pl.BlockSpec((1, cout), lambda b: (0, 0)),
            pl.BlockSpec((1, cout), lambda b: (0, 0)),
        ],
        out_specs=pl.BlockSpec((None, n, cout), lambda b: (b, 0, 0)),
        out_shape=jax.ShapeDtypeStruct((bsz, n, cout), jnp.float32),
    )(ht, h, wsum_t, wa_t, bias_row, tb_row)


def _final_kernel(mt_ref, wms_ref, bias_ref, tb_ref, ta_ref, md_ref, rd_ref,
                  w_ref, out_ref):
    emb = jnp.dot(mt_ref[...], wms_ref[...], preferred_element_type=jnp.float32)
    emb = emb + bias_ref[...]
    emb = jnp.where(emb >= 0, emb, 0.2 * emb)
    g = jnp.max(emb, axis=0, keepdims=True)
    tb = tb_ref[...]
    ta = ta_ref[...]
    md = md_ref[...]
    rd = rd_ref[...]
    m = g
    sp0 = _spike(m - tb)
    m = m * (1.0 - sp0)
    rf = sp0
    th = tb + ((tb + ta * sp0) - tb) * 0.95
    m = m * md * (1.0 - rf)
    sp1 = _spike(m - th)
    m = m * (1.0 - sp1)
    rf = rf * rd + sp1
    th = tb + ((th + ta * sp1) - tb) * 0.95
    m = m * md * (1.0 - rf)
    sp2 = _spike(m - th)
    out_ref[...] = w_ref[0] * sp0 + w_ref[1] * sp1 + w_ref[2] * sp2


def _final_layer(mt, wms_t, bias_row, tb, ta, md, rd, w3):
    bsz, n, cm = mt.shape
    e = wms_t.shape[1]
    row = lambda b: (0, 0)
    out = pl.pallas_call(
        _final_kernel,
        grid=(bsz,),
        in_specs=[
            pl.BlockSpec((None, n, cm), lambda b: (b, 0, 0)),
            pl.BlockSpec((cm, e), row),
            pl.BlockSpec((1, e), row),
            pl.BlockSpec((1, e), row),
            pl.BlockSpec((1, e), row),
            pl.BlockSpec((1, e), row),
            pl.BlockSpec((1, e), row),
            pl.BlockSpec(memory_space=pltpu.SMEM),
        ],
        out_specs=pl.BlockSpec((None, 1, e), lambda b: (b, 0, 0)),
        out_shape=jax.ShapeDtypeStruct((bsz, 1, e), jnp.float32),
    )(mt, wms_t, bias_row, tb, ta, md, rd, w3)
    return out[:, 0, :]


def kernel(x, params):
    scale = 1.0 / math.sqrt(1.0 + _BN_EPS)
    ht = jnp.transpose(x, (0, 2, 1))
    h = x
    feats = []
    for i in range(4):
        w = params['w%d' % i]
        c = w.shape[1] // 2
        sg = (params['g%d' % i] * scale)[:, None]
        wa_t = jnp.transpose(w[:, :c] * sg)
        wsum_t = jnp.transpose((w[:, :c] + w[:, c:]) * sg)
        ht = _edge_layer(ht, h, wsum_t, wa_t,
                         params['b%d' % i][None, :],
                         params['tb%d' % i][None, :])
        h = jnp.transpose(ht, (0, 2, 1))
        feats.append(ht)
    mt = jnp.concatenate(feats, axis=-1)
    wms_t = jnp.transpose(params['wms'] * (params['gms'] * scale)[:, None])
    w3 = jax.nn.softmax(params['tw'])
    return _final_layer(
        mt, wms_t,
        params['bms'][None, :],
        params['tb4'][None, :],
        jnp.clip(params['ta4'], 0.001, 0.1)[None, :],
        jnp.clip(params['md4'], 0.1, 0.99)[None, :],
        jnp.clip(params['rd4'], 0.1, 0.95)[None, :],
        w3,
    )


# speedup vs baseline: 33.0222x; 1.0762x over previous
import functools
import math

import jax
import jax.numpy as jnp
from jax import lax
from jax.experimental import pallas as pl
from jax.experimental.pallas import tpu as pltpu

_K = 20
_BN_EPS = 1e-5
_GRAD_WIDTH = 10.0
_SQRT_2PI = math.sqrt(2.0 * math.pi)
_SPIKE_PEAK = 0.4154990014554293


def _spike(v):
    xc = jnp.clip(v, -10.0, 10.0)
    gaussian = jnp.exp(-xc * xc / 2.0) / _SQRT_2PI
    sigmoid = jax.nn.sigmoid(_GRAD_WIDTH * xc)
    return 0.5 * gaussian + 0.5 * sigmoid


def _edge_kernel(ht_ref, h_ref, wsum_ref, wa_ref, bias_ref, tb_ref, out_ref, *, n, k):
    ht = ht_ref[...]
    h = h_ref[...]
    inner = -2.0 * jnp.dot(ht, h, preferred_element_type=jnp.float32)
    xx_col = jnp.sum(ht * ht, axis=1, keepdims=True)
    xx_row = jnp.sum(h * h, axis=0, keepdims=True)
    vals = -xx_col - inner - xx_row
    iota = lax.broadcasted_iota(jnp.int32, (n, n), 1)

    ut = jnp.dot(ht, wsum_ref[...], preferred_element_type=jnp.float32)
    vbt = jnp.dot(ht, wa_ref[...], preferred_element_type=jnp.float32) - bias_ref[...]
    tb = tb_ref[...]

    z = _SPIKE_PEAK + tb
    ustar = vbt + jnp.where(z >= 0, z, 5.0 * z)

    glo = None
    ghi = None
    for _ in range(k):
        mx = jnp.max(vals, axis=1, keepdims=True)
        aj = jnp.min(jnp.where(vals == mx, iota, n), axis=1, keepdims=True)
        sel = iota == aj
        vals = jnp.where(sel, -jnp.inf, vals)
        onehot = jnp.where(sel, 1.0, 0.0)
        g = jnp.dot(onehot, ut, preferred_element_type=jnp.float32)
        below = g <= ustar
        lo_j = jnp.where(below, g, -jnp.inf)
        hi_j = jnp.where(below, jnp.inf, g)
        glo = lo_j if glo is None else jnp.maximum(glo, lo_j)
        ghi = hi_j if ghi is None else jnp.minimum(ghi, hi_j)

    ylo = glo - vbt
    ylo = jnp.where(ylo >= 0, ylo, 0.2 * ylo)
    sp_lo = _spike(ylo - tb)
    yhi = ghi - vbt
    yhi = jnp.where(yhi >= 0, yhi, 0.2 * yhi)
    sp_hi = jnp.where(ghi == jnp.inf, 0.0, _spike(yhi - tb))
    out_ref[...] = jnp.maximum(sp_lo, sp_hi)


def _edge_layer(ht, h, wsum_t, wa_t, bias_row, tb_row):
    bsz, n, c = ht.shape
    cout = wsum_t.shape[1]
    return pl.pallas_call(
        functools.partial(_edge_kernel, n=n, k=_K),
        grid=(bsz,),
        in_specs=[
            pl.BlockSpec((None, n, c), lambda b: (b, 0, 0)),
            pl.BlockSpec((None, c, n), lambda b: (b, 0, 0)),
            pl.BlockSpec((c, cout), lambda b: (0, 0)),
            pl.BlockSpec((c, cout), lambda b: (0, 0)),
            pl.BlockSpec((1, cout), lambda b: (0, 0)),
            pl.BlockSpec((1, cout), lambda b: (0, 0)),
        ],
        out_specs=pl.BlockSpec((None, n, cout), lambda b: (b, 0, 0)),
        out_shape=jax.ShapeDtypeStruct((bsz, n, cout), jnp.float32),
    )(ht, h, wsum_t, wa_t, bias_row, tb_row)


def _final_kernel(mt_ref, wms_ref, bias_ref, tb_ref, ta_ref, md_ref, rd_ref,
                  w_ref, out_ref):
    emb = jnp.dot(mt_ref[...], wms_ref[...], preferred_element_type=jnp.float32)
    emb = emb + bias_ref[...]
    emb = jnp.where(emb >= 0, emb, 0.2 * emb)
    g = jnp.max(emb, axis=0, keepdims=True)
    tb = tb_ref[...]
    ta = ta_ref[...]
    md = md_ref[...]
    rd = rd_ref[...]
    m = g
    sp0 = _spike(m - tb)
    m = m * (1.0 - sp0)
    rf = sp0
    th = tb + ((tb + ta * sp0) - tb) * 0.95
    m = m * md * (1.0 - rf)
    sp1 = _spike(m - th)
    m = m * (1.0 - sp1)
    rf = rf * rd + sp1
    th = tb + ((th + ta * sp1) - tb) * 0.95
    m = m * md * (1.0 - rf)
    sp2 = _spike(m - th)
    out_ref[...] = w_ref[0] * sp0 + w_ref[1] * sp1 + w_ref[2] * sp2


def _final_layer(mt, wms_t, bias_row, tb, ta, md, rd, w3):
    bsz, n, cm = mt.shape
    e = wms_t.shape[1]
    row = lambda b: (0, 0)
    out = pl.pallas_call(
        _final_kernel,
        grid=(bsz,),
        in_specs=[
            pl.BlockSpec((None, n, cm), lambda b: (b, 0, 0)),
            pl.BlockSpec((cm, e), row),
            pl.BlockSpec((1, e), row),
            pl.BlockSpec((1, e), row),
            pl.BlockSpec((1, e), row),
            pl.BlockSpec((1, e), row),
            pl.BlockSpec((1, e), row),
            pl.BlockSpec(memory_space=pltpu.SMEM),
        ],
        out_specs=pl.BlockSpec((None, 1, e), lambda b: (b, 0, 0)),
        out_shape=jax.ShapeDtypeStruct((bsz, 1, e), jnp.float32),
    )(mt, wms_t, bias_row, tb, ta, md, rd, w3)
    return out[:, 0, :]


def kernel(x, params):
    scale = 1.0 / math.sqrt(1.0 + _BN_EPS)
    ht = jnp.transpose(x, (0, 2, 1))
    h = x
    feats = []
    for i in range(4):
        w = params['w%d' % i]
        c = w.shape[1] // 2
        sg = (params['g%d' % i] * scale)[:, None]
        wa_t = jnp.transpose(w[:, :c] * sg)
        wsum_t = jnp.transpose((w[:, :c] + w[:, c:]) * sg)
        ht = _edge_layer(ht, h, wsum_t, wa_t,
                         params['b%d' % i][None, :],
                         params['tb%d' % i][None, :])
        h = jnp.transpose(ht, (0, 2, 1))
        feats.append(ht)
    mt = jnp.concatenate(feats, axis=-1)
    wms_t = jnp.transpose(params['wms'] * (params['gms'] * scale)[:, None])
    w3 = jax.nn.softmax(params['tw'])
    return _final_layer(
        mt, wms_t,
        params['bms'][None, :],
        params['tb4'][None, :],
        jnp.clip(params['ta4'], 0.001, 0.1)[None, :],
        jnp.clip(params['md4'], 0.1, 0.99)[None, :],
        jnp.clip(params['rd4'], 0.1, 0.95)[None, :],
        w3,
    )


# speedup vs baseline: 35.1205x; 1.0635x over previous
import functools
import math

import jax
import jax.numpy as jnp
from jax import lax
from jax.experimental import pallas as pl
from jax.experimental.pallas import tpu as pltpu

_K = 20
_BN_EPS = 1e-5
_GRAD_WIDTH = 10.0
_SQRT_2PI = math.sqrt(2.0 * math.pi)
_SPIKE_PEAK = 0.4154990014554293

_NT = (((1,), (1,)), ((), ()))


def _spike(v):
    xc = jnp.clip(v, -10.0, 10.0)
    gaussian = jnp.exp(-xc * xc / 2.0) / _SQRT_2PI
    sigmoid = jax.nn.sigmoid(_GRAD_WIDTH * xc)
    return 0.5 * gaussian + 0.5 * sigmoid


def _edge_block(ht, wsum, wa, bias, tb, n, k):
    c = ht.shape[1]
    inner = -2.0 * lax.dot_general(ht, ht, _NT,
                                   preferred_element_type=jnp.float32)
    ht2 = ht * ht
    xx_row = lax.dot_general(jnp.ones((1, c), jnp.float32), ht2, _NT,
                             preferred_element_type=jnp.float32)
    vals = -inner - xx_row
    iota = lax.broadcasted_iota(jnp.int32, (n, n), 1).astype(jnp.float32)

    ut = jnp.dot(ht, wsum, preferred_element_type=jnp.float32)
    vbt = jnp.dot(ht, wa, preferred_element_type=jnp.float32) - bias
    z = _SPIKE_PEAK + tb
    ustar = vbt + jnp.where(z >= 0, z, 5.0 * z)

    glo = None
    ghi = None
    for _ in range(k):
        mx = jnp.max(vals, axis=1, keepdims=True)
        aj = jnp.min(jnp.where(vals == mx, iota, float(n)), axis=1,
                     keepdims=True)
        sel = iota == aj
        vals = jnp.where(sel, -jnp.inf, vals)
        onehot = jnp.where(sel, 1.0, 0.0)
        g = jnp.dot(onehot, ut, preferred_element_type=jnp.float32)
        below = g <= ustar
        lo_j = jnp.where(below, g, -jnp.inf)
        hi_j = jnp.where(below, jnp.inf, g)
        glo = lo_j if glo is None else jnp.maximum(glo, lo_j)
        ghi = hi_j if ghi is None else jnp.minimum(ghi, hi_j)

    ylo = glo - vbt
    ylo = jnp.where(ylo >= 0, ylo, 0.2 * ylo)
    sp_lo = _spike(ylo - tb)
    yhi = ghi - vbt
    yhi = jnp.where(yhi >= 0, yhi, 0.2 * yhi)
    sp_hi = jnp.where(ghi == jnp.inf, 0.0, _spike(yhi - tb))
    return jnp.maximum(sp_lo, sp_hi)


def _net_kernel(ht_ref,
                ws0_ref, wa0_ref, b0_ref, tb0_ref,
                ws1_ref, wa1_ref, b1_ref, tb1_ref,
                ws2_ref, wa2_ref, b2_ref, tb2_ref,
                ws3_ref, wa3_ref, b3_ref, tb3_ref,
                wms_ref, bms_ref, tb4_ref, ta4_ref, md4_ref, rd4_ref,
                w_ref, out_ref, *, n, k):
    layer_refs = [
        (ws0_ref, wa0_ref, b0_ref, tb0_ref),
        (ws1_ref, wa1_ref, b1_ref, tb1_ref),
        (ws2_ref, wa2_ref, b2_ref, tb2_ref),
        (ws3_ref, wa3_ref, b3_ref, tb3_ref),
    ]
    ht = ht_ref[...]
    feats = []
    for ws_ref, wa_ref, b_ref, tbl_ref in layer_refs:
        ht = _edge_block(ht, ws_ref[...], wa_ref[...], b_ref[...],
                         tbl_ref[...], n, k)
        feats.append(ht)
    mt = jnp.concatenate(feats, axis=-1)

    emb = jnp.dot(mt, wms_ref[...], preferred_element_type=jnp.float32)
    emb = emb + bms_ref[...]
    emb = jnp.where(emb >= 0, emb, 0.2 * emb)
    gp = jnp.max(emb, axis=0, keepdims=True)

    tb = tb4_ref[...]
    ta = ta4_ref[...]
    md = md4_ref[...]
    rd = rd4_ref[...]
    m = gp
    sp0 = _spike(m - tb)
    m = m * (1.0 - sp0)
    rf = sp0
    th = tb + ((tb + ta * sp0) - tb) * 0.95
    m = m * md * (1.0 - rf)
    sp1 = _spike(m - th)
    m = m * (1.0 - sp1)
    rf = rf * rd + sp1
    th = tb + ((th + ta * sp1) - tb) * 0.95
    m = m * md * (1.0 - rf)
    sp2 = _spike(m - th)
    out_ref[...] = w_ref[0] * sp0 + w_ref[1] * sp1 + w_ref[2] * sp2


def kernel(x, params):
    bsz, cin, n = x.shape
    e = params['wms'].shape[0]
    scale = 1.0 / math.sqrt(1.0 + _BN_EPS)
    ht0 = jnp.transpose(x, (0, 2, 1))

    full = lambda shape: pl.BlockSpec(shape, lambda b: tuple(0 for _ in shape))
    in_specs = [pl.BlockSpec((None, n, cin), lambda b: (b, 0, 0))]
    args = [ht0]
    for i in range(4):
        w = params['w%d' % i]
        c = w.shape[1] // 2
        cout = w.shape[0]
        sg = (params['g%d' % i] * scale)[:, None]
        wa_t = jnp.transpose(w[:, :c] * sg)
        wsum_t = jnp.transpose((w[:, :c] + w[:, c:]) * sg)
        args += [wsum_t, wa_t, params['b%d' % i][None, :],
                 params['tb%d' % i][None, :]]
        in_specs += [full((c, cout)), full((c, cout)),
                     full((1, cout)), full((1, cout))]

    wms_t = jnp.transpose(params['wms'] * (params['gms'] * scale)[:, None])
    args += [wms_t, params['bms'][None, :],
             params['tb4'][None, :],
             jnp.clip(params['ta4'], 0.001, 0.1)[None, :],
             jnp.clip(params['md4'], 0.1, 0.99)[None, :],
             jnp.clip(params['rd4'], 0.1, 0.95)[None, :],
             jax.nn.softmax(params['tw'])]
    in_specs += [full((wms_t.shape[0], e)), full((1, e)), full((1, e)),
                 full((1, e)), full((1, e)), full((1, e)),
                 pl.BlockSpec(memory_space=pltpu.SMEM)]

    out = pl.pallas_call(
        functools.partial(_net_kernel, n=n, k=_K),
        grid=(bsz,),
        in_specs=in_specs,
        out_specs=pl.BlockSpec((None, 1, e), lambda b: (b, 0, 0)),
        out_shape=jax.ShapeDtypeStruct((bsz, 1, e), jnp.float32),
    )(*args)
    return out[:, 0, :]


# speedup vs baseline: 43.3487x; 1.2343x over previous
import functools
import math

import jax
import jax.numpy as jnp
from jax import lax
from jax.experimental import pallas as pl
from jax.experimental.pallas import tpu as pltpu

_K = 20
_BN_EPS = 1e-5
_GRAD_WIDTH = 10.0
_SQRT_2PI = math.sqrt(2.0 * math.pi)
_SPIKE_PEAK = 0.4154990014554293

_NT = (((1,), (1,)), ((), ()))


def _spike(v):
    xc = jnp.clip(v, -10.0, 10.0)
    gaussian = jnp.exp(-xc * xc / 2.0) / _SQRT_2PI
    sigmoid = jax.nn.sigmoid(_GRAD_WIDTH * xc)
    return 0.5 * gaussian + 0.5 * sigmoid


def _edge_block(ht, wsum, wa, bias, tb, n, k):
    c = ht.shape[1]
    inner = -2.0 * lax.dot_general(ht, ht, _NT,
                                   preferred_element_type=jnp.float32)
    ht2 = ht * ht
    xx_row = lax.dot_general(jnp.ones((1, c), jnp.float32), ht2, _NT,
                             preferred_element_type=jnp.float32)
    vals = -inner - xx_row
    iota = lax.broadcasted_iota(jnp.int32, (n, n), 1).astype(jnp.float32)

    ut = jnp.dot(ht, wsum, preferred_element_type=jnp.float32)
    vbt = jnp.dot(ht, wa, preferred_element_type=jnp.float32) - bias
    z = _SPIKE_PEAK + tb
    ustar = vbt + jnp.where(z >= 0, z, 5.0 * z)

    glo = None
    ghi = None
    for _ in range(k):
        mx = jnp.max(vals, axis=1, keepdims=True)
        aj = jnp.min(jnp.where(vals == mx, iota, float(n)), axis=1,
                     keepdims=True)
        sel = iota == aj
        vals = jnp.where(sel, -jnp.inf, vals)
        onehot = jnp.where(sel, 1.0, 0.0)
        g = jnp.dot(onehot, ut, preferred_element_type=jnp.float32)
        below = g <= ustar
        lo_j = jnp.where(below, g, -jnp.inf)
        hi_j = jnp.where(below, jnp.inf, g)
        glo = lo_j if glo is None else jnp.maximum(glo, lo_j)
        ghi = hi_j if ghi is None else jnp.minimum(ghi, hi_j)

    ylo = glo - vbt
    ylo = jnp.where(ylo >= 0, ylo, 0.2 * ylo)
    sp_lo = _spike(ylo - tb)
    yhi = ghi - vbt
    yhi = jnp.where(yhi >= 0, yhi, 0.2 * yhi)
    sp_hi = jnp.where(ghi == jnp.inf, 0.0, _spike(yhi - tb))
    return jnp.maximum(sp_lo, sp_hi)


def _net_kernel(ht_ref,
                ws0_ref, wa0_ref, b0_ref, tb0_ref,
                ws1_ref, wa1_ref, b1_ref, tb1_ref,
                ws2_ref, wa2_ref, b2_ref, tb2_ref,
                ws3_ref, wa3_ref, b3_ref, tb3_ref,
                wms_ref, bms_ref, tb4_ref, ta4_ref, md4_ref, rd4_ref,
                w_ref, out_ref, *, n, k, bsz):
    layer_refs = [
        (ws0_ref, wa0_ref, b0_ref, tb0_ref),
        (ws1_ref, wa1_ref, b1_ref, tb1_ref),
        (ws2_ref, wa2_ref, b2_ref, tb2_ref),
        (ws3_ref, wa3_ref, b3_ref, tb3_ref),
    ]
    tb = tb4_ref[...]
    ta = ta4_ref[...]
    md = md4_ref[...]
    rd = rd4_ref[...]
    for b in range(bsz):
        ht = ht_ref[b]
        feats = []
        for ws_ref, wa_ref, b_ref, tbl_ref in layer_refs:
            ht = _edge_block(ht, ws_ref[...], wa_ref[...], b_ref[...],
                             tbl_ref[...], n, k)
            feats.append(ht)
        mt = jnp.concatenate(feats, axis=-1)

        emb = jnp.dot(mt, wms_ref[...], preferred_element_type=jnp.float32)
        emb = emb + bms_ref[...]
        emb = jnp.where(emb >= 0, emb, 0.2 * emb)
        gp = jnp.max(emb, axis=0, keepdims=True)

        m = gp
        sp0 = _spike(m - tb)
        m = m * (1.0 - sp0)
        rf = sp0
        th = tb + ((tb + ta * sp0) - tb) * 0.95
        m = m * md * (1.0 - rf)
        sp1 = _spike(m - th)
        m = m * (1.0 - sp1)
        rf = rf * rd + sp1
        th = tb + ((th + ta * sp1) - tb) * 0.95
        m = m * md * (1.0 - rf)
        sp2 = _spike(m - th)
        out_ref[b] = w_ref[0] * sp0 + w_ref[1] * sp1 + w_ref[2] * sp2


def kernel(x, params):
    bsz, cin, n = x.shape
    e = params['wms'].shape[0]
    scale = 1.0 / math.sqrt(1.0 + _BN_EPS)
    ht0 = jnp.transpose(x, (0, 2, 1))

    full = lambda shape: pl.BlockSpec(shape, lambda: tuple(0 for _ in shape))
    in_specs = [pl.BlockSpec((bsz, n, cin), lambda: (0, 0, 0))]
    args = [ht0]
    for i in range(4):
        w = params['w%d' % i]
        c = w.shape[1] // 2
        cout = w.shape[0]
        sg = (params['g%d' % i] * scale)[:, None]
        wa_t = jnp.transpose(w[:, :c] * sg)
        wsum_t = jnp.transpose((w[:, :c] + w[:, c:]) * sg)
        args += [wsum_t, wa_t, params['b%d' % i][None, :],
                 params['tb%d' % i][None, :]]
        in_specs += [full((c, cout)), full((c, cout)),
                     full((1, cout)), full((1, cout))]

    wms_t = jnp.transpose(params['wms'] * (params['gms'] * scale)[:, None])
    args += [wms_t, params['bms'][None, :],
             params['tb4'][None, :],
             jnp.clip(params['ta4'], 0.001, 0.1)[None, :],
             jnp.clip(params['md4'], 0.1, 0.99)[None, :],
             jnp.clip(params['rd4'], 0.1, 0.95)[None, :],
             jax.nn.softmax(params['tw'])]
    in_specs += [full((wms_t.shape[0], e)), full((1, e)), full((1, e)),
                 full((1, e)), full((1, e)), full((1, e)),
                 pl.BlockSpec(memory_space=pltpu.SMEM)]

    out = pl.pallas_call(
        functools.partial(_net_kernel, n=n, k=_K, bsz=bsz),
        grid=(),
        in_specs=in_specs,
        out_specs=pl.BlockSpec((bsz, 1, e), lambda: (0, 0, 0)),
        out_shape=jax.ShapeDtypeStruct((bsz, 1, e), jnp.float32),
    )(*args)
    return out[:, 0, :]
